# baseline (speedup 1.0000x reference)
import jax
import jax.numpy as jnp
from jax.experimental import pallas as pl
from jax.experimental.pallas import tpu as pltpu

_TILE = 128
_STRIP = 1152


def _round_up(v, m):
    return ((v + m - 1) // m) * m


def _embed_kernel(x_ref, w_ref, b_ref, ohi_ref):
    y = jnp.dot(x_ref[...], w_ref[...], preferred_element_type=jnp.float32)
    h = jnp.maximum(y + b_ref[...], 0.0)
    ohi_ref[...] = h.astype(jnp.bfloat16)


def _embed(x, w, b):
    n_pad, d = x.shape
    h_dim = w.shape[1]
    gi = n_pad // _STRIP
    return pl.pallas_call(
        _embed_kernel,
        out_shape=jax.ShapeDtypeStruct((n_pad, h_dim), jnp.bfloat16),
        grid=(gi,),
        in_specs=[
            pl.BlockSpec((_STRIP, d), lambda i: (i, 0)),
            pl.BlockSpec((d, h_dim), lambda i: (0, 0)),
            pl.BlockSpec((1, h_dim), lambda i: (0, 0)),
        ],
        out_specs=pl.BlockSpec((_STRIP, h_dim), lambda i: (i, 0)),
        compiler_params=pltpu.CompilerParams(
            dimension_semantics=("parallel",)),
    )(x, w, b)


def _fused_kernel(h0_ref, a_ref, inv_ref,
                  wl_ref, wr_ref, b_ref, o_ref, hc_ref, hn_ref):
    l = pl.program_id(0)
    i = pl.program_id(1)
    ni = pl.num_programs(1)
    strip = hn_ref.shape[0] // ni
    i0 = pl.multiple_of(i * strip, strip)

    @pl.when((l == 0) & (i == 0))
    def _():
        hc_ref[...] = h0_ref[...]

    agg = jnp.dot(a_ref[...], hc_ref[...],
                  preferred_element_type=jnp.float32)
    agg = agg * inv_ref[:, 0:1]
    h_self = hc_ref[pl.ds(i0, strip), :]
    y = jnp.dot(agg.astype(jnp.bfloat16), wl_ref[0],
                preferred_element_type=jnp.float32)
    y = y + jnp.dot(h_self, wr_ref[0],
                    preferred_element_type=jnp.float32)
    h = jnp.maximum(y + b_ref[0], 0.0)
    hn_ref[pl.ds(i0, strip), :] = h.astype(jnp.bfloat16)
    o_ref[...] = h

    @pl.when(i == ni - 1)
    def _():
        hc_ref[...] = hn_ref[...]


def _fused_forward(h0, adj, inv_mat, wl_cat, wr_cat, b_cat):
    n_pad, h_dim = h0.shape
    num_layers = wl_cat.shape[0]
    gi = n_pad // _STRIP
    nl = num_layers

    return pl.pallas_call(
        _fused_kernel,
        out_shape=jax.ShapeDtypeStruct((n_pad, h_dim), jnp.float32),
        grid=(num_layers, gi),
        in_specs=[
            pl.BlockSpec((n_pad, h_dim), lambda l, i: (0, 0)),
            pl.BlockSpec((_STRIP, n_pad), lambda l, i: (i, 0)),
            pl.BlockSpec((_STRIP, _TILE), lambda l, i: (i, 0)),
            pl.BlockSpec((1, h_dim, h_dim), lambda l, i: (l, 0, 0)),
            pl.BlockSpec((1, h_dim, h_dim), lambda l, i: (l, 0, 0)),
            pl.BlockSpec((1, 1, h_dim), lambda l, i: (l, 0, 0)),
        ],
        out_specs=pl.BlockSpec((_STRIP, h_dim),
                               lambda l, i: (jnp.where(l == nl - 1, i, 0), 0)),
        scratch_shapes=[
            pltpu.VMEM((n_pad, h_dim), jnp.bfloat16),
            pltpu.VMEM((n_pad, h_dim), jnp.bfloat16),
        ],
        compiler_params=pltpu.CompilerParams(
            dimension_semantics=("arbitrary", "arbitrary")),
    )(h0, adj, inv_mat, wl_cat, wr_cat, b_cat)


def kernel(emb_w, emb_b, conv_wl, conv_bl, conv_wr, x, edge_index):
    n, d_in = x.shape
    hidden = emb_w.shape[0]
    num_layers = conv_wl.shape[0]
    n_pad = _round_up(n, _TILE)

    x_pad = jnp.pad(x, ((0, n_pad - n), (0, 0)))

    src, dst = edge_index[0], edge_index[1]
    adj32 = jnp.zeros((n_pad, n_pad), jnp.float32)
    adj32 = adj32.at[dst, src].add(1.0)
    adj = adj32.astype(jnp.bfloat16)
    deg = jnp.zeros((n_pad,), jnp.float32).at[dst].add(1.0)
    inv = 1.0 / jnp.maximum(deg, 1.0)
    inv_mat = jnp.broadcast_to(inv[:, None], (n_pad, _TILE))

    wl_cat = jnp.transpose(conv_wl, (0, 2, 1)).astype(jnp.bfloat16)
    wr_cat = jnp.transpose(conv_wr, (0, 2, 1)).astype(jnp.bfloat16)

    h0 = _embed(x_pad, emb_w.T, emb_b)
    h32 = _fused_forward(h0, adj, inv_mat, wl_cat, wr_cat, conv_bl)
    return h32[:n, :hidden]


# speedup vs baseline: 1.0027x; 1.0027x over previous
import jax
import jax.numpy as jnp
from jax.experimental import pallas as pl
from jax.experimental.pallas import tpu as pltpu

_TILE = 128
_STRIP = 1152


def _round_up(v, m):
    return ((v + m - 1) // m) * m


def _fused_kernel(x_ref, we_ref, be_ref, a_ref, inv_ref,
                  wl_ref, wr_ref, b_ref, o_ref, hc_ref, hn_ref):
    l = pl.program_id(0)
    i = pl.program_id(1)
    ni = pl.num_programs(1)
    strip = hn_ref.shape[0] // ni
    i0 = pl.multiple_of(i * strip, strip)

    @pl.when(l == 0)
    def _():
        y = jnp.dot(x_ref[...], we_ref[...],
                    preferred_element_type=jnp.float32)
        h = jnp.maximum(y + be_ref[...], 0.0)
        hn_ref[pl.ds(i0, strip), :] = h.astype(jnp.bfloat16)

    @pl.when(l > 0)
    def _():
        agg = jnp.dot(a_ref[...], hc_ref[...],
                      preferred_element_type=jnp.float32)
        agg = agg * inv_ref[:, 0:1]
        h_self = hc_ref[pl.ds(i0, strip), :]
        y = jnp.dot(agg.astype(jnp.bfloat16), wl_ref[0],
                    preferred_element_type=jnp.float32)
        y = y + jnp.dot(h_self, wr_ref[0],
                        preferred_element_type=jnp.float32)
        h = jnp.maximum(y + b_ref[0], 0.0)
        hn_ref[pl.ds(i0, strip), :] = h.astype(jnp.bfloat16)
        o_ref[...] = h

    @pl.when(i == ni - 1)
    def _():
        hc_ref[...] = hn_ref[...]


def _fused_forward(x_pad, weT, be, adj, inv_mat, wl_cat, wr_cat, b_cat):
    n_pad, h_dim = adj.shape[0], weT.shape[1]
    num_layers = wl_cat.shape[0]
    gi = n_pad // _STRIP
    nl = num_layers

    return pl.pallas_call(
        _fused_kernel,
        out_shape=jax.ShapeDtypeStruct((n_pad, h_dim), jnp.float32),
        grid=(num_layers + 1, gi),
        in_specs=[
            pl.BlockSpec((_STRIP, x_pad.shape[1]),
                         lambda l, i: (jnp.where(l == 0, i, gi - 1), 0)),
            pl.BlockSpec((x_pad.shape[1], h_dim), lambda l, i: (0, 0)),
            pl.BlockSpec((1, h_dim), lambda l, i: (0, 0)),
            pl.BlockSpec((_STRIP, n_pad),
                         lambda l, i: (jnp.where(l > 0, i, 0), 0)),
            pl.BlockSpec((_STRIP, _TILE),
                         lambda l, i: (jnp.where(l > 0, i, 0), 0)),
            pl.BlockSpec((1, h_dim, h_dim),
                         lambda l, i: (jnp.where(l > 0, l - 1, 0), 0, 0)),
            pl.BlockSpec((1, h_dim, h_dim),
                         lambda l, i: (jnp.where(l > 0, l - 1, 0), 0, 0)),
            pl.BlockSpec((1, 1, h_dim),
                         lambda l, i: (jnp.where(l > 0, l - 1, 0), 0, 0)),
        ],
        out_specs=pl.BlockSpec((_STRIP, h_dim),
                               lambda l, i: (jnp.where(l == nl, i, 0), 0)),
        scratch_shapes=[
            pltpu.VMEM((n_pad, h_dim), jnp.bfloat16),
            pltpu.VMEM((n_pad, h_dim), jnp.bfloat16),
        ],
        compiler_params=pltpu.CompilerParams(
            dimension_semantics=("arbitrary", "arbitrary")),
    )(x_pad, weT, be, adj, inv_mat, wl_cat, wr_cat, b_cat)


def kernel(emb_w, emb_b, conv_wl, conv_bl, conv_wr, x, edge_index):
    n, d_in = x.shape
    hidden = emb_w.shape[0]
    num_layers = conv_wl.shape[0]
    n_pad = _round_up(n, _TILE)

    x_pad = jnp.pad(x, ((0, n_pad - n), (0, 0)))

    src, dst = edge_index[0], edge_index[1]
    adj32 = jnp.zeros((n_pad, n_pad), jnp.float32)
    adj32 = adj32.at[dst, src].add(1.0)
    adj = adj32.astype(jnp.bfloat16)
    deg = jnp.zeros((n_pad,), jnp.float32).at[dst].add(1.0)
    inv = 1.0 / jnp.maximum(deg, 1.0)
    inv_mat = jnp.broadcast_to(inv[:, None], (n_pad, _TILE))

    wl_cat = jnp.transpose(conv_wl, (0, 2, 1)).astype(jnp.bfloat16)
    wr_cat = jnp.transpose(conv_wr, (0, 2, 1)).astype(jnp.bfloat16)

    h32 = _fused_forward(x_pad, emb_w.T, emb_b, adj, inv_mat,
                         wl_cat, wr_cat, conv_bl)
    return h32[:n, :hidden]
